# Initial kernel scaffold; baseline (speedup 1.0000x reference)
import jax
import jax.numpy as jnp
from jax.experimental import pallas as pl


def kernel(logits, k):
    raise NotImplementedError("write your pallas kernel here")


# speedup vs baseline: 31.2112x; 31.2112x over previous
import functools

import jax
import jax.numpy as jnp
from jax import lax
from jax.experimental import pallas as pl
from jax.experimental.pallas import tpu as pltpu
from jax.experimental.pallas import tpu_sc as plsc

L = 16
B, V = 64, 100000
NCHUNK = V // L
NBINS = 8192
NBIN_CH = NBINS // L
CAP = 512
CAP_CH = CAP // L
T = 64
ZBUF = 12800
NEG = -3.0e38
POS = 3.0e38
TOP_P = 0.9

_GDN = lax.GatherDimensionNumbers(
    offset_dims=(), collapsed_slice_dims=(0,), start_index_map=(0,))


def _permute(x, idx):
  return lax.gather(x, idx[:, None], _GDN, (1,),
                    mode=lax.GatherScatterMode.PROMISE_IN_BOUNDS)


def _splat(x_scalar):
  return jnp.full((L,), x_scalar)


def _sortkey(v):
  u = lax.bitcast_convert_type(v, jnp.uint32)
  s = lax.bitcast_convert_type(v, jnp.int32) >> 31
  flip = lax.bitcast_convert_type(s, jnp.uint32) | jnp.uint32(0x80000000)
  return u ^ flip


def _body(logits_hbm, kvec_hbm, out_hbm, in_v, hist_v, cand_v, zero_v, kv_v):
  wid = lax.axis_index("s") * 2 + lax.axis_index("c")
  iota = lax.iota(jnp.int32, L)
  pltpu.sync_copy(kvec_hbm, kv_v)
  kk = kv_v[...]

  def zinit(i, _):
    zero_v[pl.ds(i * L, L)] = jnp.zeros((L,), jnp.float32)
    return 0
  lax.fori_loop(0, ZBUF // L, zinit, 0)

  def do_row(r, _):
    row = wid * 2 + r
    pltpu.sync_copy(logits_hbm.at[pl.ds(row * V, V)], in_v)

    def hinit(i, _):
      hist_v[pl.ds(i * L, L)] = jnp.zeros((L,), jnp.int32)
      return 0
    lax.fori_loop(0, NBIN_CH, hinit, 0)

    def cinit(i, _):
      cand_v[pl.ds(i * L, L)] = jnp.full((L,), NEG, jnp.float32)
      return 0
    lax.fori_loop(0, CAP_CH, cinit, 0)

    ones = jnp.ones((L,), jnp.int32)
    def hist_body(i, _):
      v = in_v[pl.ds(i * L, L)]
      bkt = (_sortkey(v) >> jnp.uint32(19)).astype(jnp.int32)
      plsc.addupdate_scatter(hist_v, [bkt], ones)
      return 0
    lax.fori_loop(0, NCHUNK, hist_body, 0)

    def bscan(j, carry):
      acc, bstar, found = carry
      c = NBIN_CH - 1 - j
      h = hist_v[pl.ds(c * L, L)]
      tot = jnp.sum(h)
      incl = plsc.cumsum(h)
      ssum = (tot - incl) + h
      qual = (acc + ssum) >= kk
      nq = jnp.sum(qual.astype(jnp.int32))
      found_here = jnp.logical_and(jnp.logical_not(found), nq > 0)
      bstar = jnp.where(found_here, c * L + nq - 1, bstar)
      found = jnp.logical_or(found, nq > 0)
      return acc + tot, bstar, found
    _, bstar, _ = lax.fori_loop(
        0, NBIN_CH, bscan,
        (jnp.int32(0), jnp.int32(0), jnp.bool_(False)))
    bstar_v = _splat(bstar)

    def coll_body(i, cnt):
      v = in_v[pl.ds(i * L, L)]
      bkt = (_sortkey(v) >> jnp.uint32(19)).astype(jnp.int32)
      m = bkt >= bstar_v
      m = jnp.logical_and(m, cnt < (CAP - L))
      mi = m.astype(jnp.int32)
      pos = cnt + plsc.cumsum(mi) - 1
      plsc.store_scatter(cand_v, [pos], v, mask=m)
      return cnt + _splat(jnp.sum(mi))
    cnt = lax.fori_loop(0, NCHUNK, coll_body, jnp.zeros((L,), jnp.int32))
    n_ch = (jnp.max(cnt) + (L - 1)) // L

    def sel_round(rr, svecs):
      def mx_body(c, mx):
        return jnp.maximum(mx, cand_v[pl.ds(c * L, L)])
      mx = lax.fori_loop(0, n_ch, mx_body, jnp.full((L,), NEG, jnp.float32))
      g = jnp.max(mx)
      g_v = _splat(g)

      def rm_body(c, removed):
        ch = cand_v[pl.ds(c * L, L)]
        eq = ch == g_v
        any_eq = jnp.any(eq)
        ffs = plsc.all_reduce_ffs(eq)
        do = jnp.logical_and(any_eq, jnp.logical_not(removed))
        hit = jnp.logical_and(iota == ffs, _splat(do))
        cand_v[pl.ds(c * L, L)] = jnp.where(hit, _splat(NEG), ch)
        return jnp.logical_or(removed, any_eq)
      lax.fori_loop(0, n_ch, rm_body, jnp.bool_(False))

      out = []
      for c4 in range(T // L):
        sel = jnp.logical_and(_splat((rr >> 4) == c4), iota == (rr & (L - 1)))
        out.append(jnp.where(sel, g_v, svecs[c4]))
      return tuple(out)
    s = lax.fori_loop(0, T, sel_round,
                      tuple(jnp.full((L,), NEG, jnp.float32)
                            for _ in range(T // L)))

    shift_idx = (iota + (L - 1)) & (L - 1)
    lane0 = iota == 0
    kept0 = []
    carry_val = POS
    carry_f = jnp.int32(-1)
    for c4 in range(T // L):
      sc = s[c4]
      shifted = jnp.where(lane0, _splat(carry_val), _permute(sc, shift_idx))
      newseg = sc != shifted
      a = jnp.where(newseg, iota + c4 * L, -1)
      f = jnp.maximum(plsc.cummax(a), _splat(carry_f))
      kept0.append(f < kk)
      carry_val = jnp.sum(jnp.where(iota == L - 1, sc, 0.0))
      carry_f = jnp.max(f)

    mx0 = jnp.sum(jnp.where(lane0, s[0], 0.0))
    e = [jnp.where(kept0[c], jnp.exp(jnp.where(kept0[c], s[c], mx0) - mx0),
                   0.0) for c in range(T // L)]
    den1 = jnp.float32(0.0)
    for c in range(T // L):
      den1 = den1 + jnp.sum(e[c])
    p = [e[c] / den1 for c in range(T // L)]

    cums, carry = [], jnp.float32(0.0)
    for c in range(T // L):
      cc = plsc.cumsum(p[c]) + carry
      cums.append(cc)
      carry = carry + jnp.sum(p[c])

    kept2, cprev = [], jnp.float32(0.0)
    for c in range(T // L):
      prevcum = jnp.where(lane0, _splat(cprev), _permute(cums[c], shift_idx))
      kept2.append(jnp.logical_and(kept0[c],
                                   prevcum <= jnp.float32(1.0 - TOP_P)))
      cprev = jnp.sum(jnp.where(iota == L - 1, cums[c], 0.0))

    den2 = jnp.float32(0.0)
    for c in range(T // L):
      den2 = den2 + jnp.sum(jnp.where(kept2[c], p[c], 0.0))
    for c in range(T // L):
      zero_v[pl.ds(c * L, L)] = jnp.where(kept2[c], p[c] / den2, 0.0)

    rbase = row * V
    pltpu.sync_copy(zero_v, out_hbm.at[pl.ds(rbase, ZBUF)])
    for c in range(T // L):
      zero_v[pl.ds(c * L, L)] = jnp.zeros((L,), jnp.float32)
    for t in range(1, V // ZBUF + 1):
      n = min(ZBUF, V - t * ZBUF)
      pltpu.sync_copy(zero_v.at[pl.ds(0, n)],
                      out_hbm.at[pl.ds(rbase + t * ZBUF, n)])
    return 0

  lax.fori_loop(0, B // 32, do_row, 0)


@jax.jit
def _sc_call(logits, kvec):
  f = pl.kernel(
      _body,
      out_type=jax.ShapeDtypeStruct((B * V,), jnp.float32),
      mesh=plsc.VectorSubcoreMesh(core_axis_name="c", subcore_axis_name="s"),
      compiler_params=pltpu.CompilerParams(needs_layout_passes=False),
      scratch_types=[
          pltpu.VMEM((V,), jnp.float32),
          pltpu.VMEM((NBINS,), jnp.int32),
          pltpu.VMEM((CAP,), jnp.float32),
          pltpu.VMEM((ZBUF,), jnp.float32),
          pltpu.VMEM((L,), jnp.int32),
      ],
  )
  return f(logits.reshape(B * V), kvec).reshape(B, V)


def kernel(logits, k):
  kvec = jnp.full((L,), k, dtype=jnp.int32)
  return _sc_call(logits, kvec)
